# baseline (speedup 1.0000x reference)
import functools

import jax
import jax.numpy as jnp
from jax import lax
from jax.experimental import pallas as pl
from jax.experimental.pallas import tpu as pltpu
from jax.experimental.pallas import tpu_sc as plsc

EPS = 1e-5
NC = 2
NS = 16
NW = NC * NS
CH = 128


def _sc_gather(node_rep, src2, Ep):
    nchunks = Ep // (NW * CH)
    ngroups = nchunks // 8
    mesh = plsc.VectorSubcoreMesh(core_axis_name="c", subcore_axis_name="s")

    @functools.partial(
        pl.kernel,
        out_type=jax.ShapeDtypeStruct((Ep // 8, 128), jnp.float32),
        mesh=mesh,
        scratch_types=[
            pltpu.VMEM((8, CH), jnp.int32),
            pltpu.VMEM((CH, 128), jnp.float32),
            pltpu.VMEM((CH, 128), jnp.float32),
            pltpu.VMEM((CH, 128), jnp.float32),
            pltpu.VMEM((CH, 128), jnp.float32),
            pltpu.VMEM((CH // 8, 128), jnp.float32),
            pltpu.VMEM((CH // 8, 128), jnp.float32),
            pltpu.SemaphoreType.DMA,
            pltpu.SemaphoreType.DMA,
            pltpu.SemaphoreType.DMA,
            pltpu.SemaphoreType.DMA,
            pltpu.SemaphoreType.DMA,
            pltpu.SemaphoreType.DMA,
        ],
    )
    def k(node_hbm, src_hbm, out_hbm, idx_v, r0, r1, r2, r3, x0, x1,
          sg0, sg1, sg2, sg3, so0, so1):
        c = lax.axis_index("c")
        s = lax.axis_index("s")
        wid = s * NC + c
        rows = (r0, r1, r2, r3)
        xb = (x0, x1)
        sg = (sg0, sg1, sg2, sg3)
        so = (so0, so1)

        def body(j, carry):
            gg = wid * ngroups + j
            pltpu.sync_copy(src_hbm.at[pl.ds(gg * 8, 8)], idx_v)
            for kk in range(3):
                pltpu.async_copy(node_hbm.at[idx_v.at[kk]],
                                 rows[kk], sg[kk])
            for kk in range(8):
                sl = kk % 4
                xs = kk % 2
                if kk < 5:
                    pltpu.async_copy(node_hbm.at[idx_v.at[kk + 3]],
                                     rows[(kk + 3) % 4], sg[(kk + 3) % 4])
                pltpu.make_async_copy(node_hbm.at[idx_v.at[kk]],
                                      rows[sl], sg[sl]).wait()
                if kk >= 2:
                    g2 = gg * 8 + kk - 2
                    pltpu.make_async_copy(
                        xb[xs],
                        out_hbm.at[pl.ds(g2 * (CH // 8), CH // 8)],
                        so[xs]).wait()
                for q in range(CH):
                    xb[xs][q // 8, pl.ds(16 * (q % 8), 16)] = \
                        rows[sl][q, pl.ds(0, 16)]
                g = gg * 8 + kk
                pltpu.async_copy(xb[xs],
                                 out_hbm.at[pl.ds(g * (CH // 8), CH // 8)],
                                 so[xs])
            for kk in range(6, 8):
                xs = kk % 2
                g = gg * 8 + kk
                pltpu.make_async_copy(
                    xb[xs], out_hbm.at[pl.ds(g * (CH // 8), CH // 8)],
                    so[xs]).wait()
            return carry

        lax.fori_loop(0, ngroups, body, 0)

    return k(node_rep, src2)


def _sc_scatter(msg_pk, dst2, Ep, N2):
    nchunks = Ep // (NW * CH)
    ngroups = nchunks // 8
    rows_per = N2 // NS
    ZR = 79
    mesh = plsc.VectorSubcoreMesh(core_axis_name="c", subcore_axis_name="s")

    @functools.partial(
        pl.kernel,
        out_type=jax.ShapeDtypeStruct((NC * N2, 128), jnp.float32),
        mesh=mesh,
        scratch_types=[
            pltpu.VMEM_SHARED((N2, 128), jnp.float32),
            pltpu.VMEM((8, CH), jnp.int32),
            pltpu.VMEM((CH // 8, 128), jnp.float32),
            pltpu.VMEM((CH // 8, 128), jnp.float32),
            pltpu.VMEM((CH, 128), jnp.float32),
            pltpu.VMEM((CH, 128), jnp.float32),
            pltpu.VMEM((ZR, 128), jnp.float32),
            pltpu.SemaphoreType.DMA,
            pltpu.SemaphoreType.DMA,
            pltpu.SemaphoreType.DMA,
            pltpu.SemaphoreType.DMA,
        ],
    )
    def k(msg_hbm, dst_hbm, out_hbm, acc, idx_v, m0, m1, b0, b1, zer,
          sm0, sm1, ss0, ss1):
        c = lax.axis_index("c")
        s = lax.axis_index("s")
        wid = s * NC + c
        mb = (m0, m1)
        buf = (b0, b1)
        sm = (sm0, sm1)
        ss = (ss0, ss1)

        zv = jnp.zeros((16,), jnp.float32)
        ov = jnp.ones((16,), jnp.float32)

        def fz(i, carry):
            for t in range(8):
                zer[i, pl.ds(16 * t, 16)] = zv
            return carry

        lax.fori_loop(0, ZR, fz, 0)

        def fb(i, carry):
            for b in buf:
                b[i, pl.ds(16, 16)] = ov
                for t in range(2, 8):
                    b[i, pl.ds(16 * t, 16)] = zv
            return carry

        lax.fori_loop(0, CH, fb, 0)

        nz = (rows_per + ZR - 1) // ZR

        def za(i, carry):
            r = jnp.minimum(i * ZR, rows_per - ZR)
            pltpu.sync_copy(zer, acc.at[pl.ds(s * rows_per + r, ZR)])
            return carry

        lax.fori_loop(0, nz, za, 0)
        plsc.subcore_barrier()

        def body(j, carry):
            gg = wid * ngroups + j
            pltpu.sync_copy(dst_hbm.at[pl.ds(gg * 8, 8)], idx_v)
            g0 = gg * 8
            pltpu.async_copy(msg_hbm.at[pl.ds(g0 * (CH // 8), CH // 8)],
                             mb[0], sm[0])
            for kk in range(8):
                sl = kk % 2
                g = gg * 8 + kk
                if kk < 7:
                    g1 = g + 1
                    pltpu.async_copy(
                        msg_hbm.at[pl.ds(g1 * (CH // 8), CH // 8)],
                        mb[(kk + 1) % 2], sm[(kk + 1) % 2])
                pltpu.make_async_copy(
                    msg_hbm.at[pl.ds(g * (CH // 8), CH // 8)],
                    mb[sl], sm[sl]).wait()
                if kk >= 2:
                    pltpu.make_async_copy(buf[sl], acc.at[idx_v.at[kk - 2]],
                                          ss[sl]).wait()
                for q in range(CH):
                    buf[sl][q, pl.ds(0, 16)] = \
                        mb[sl][q // 8, pl.ds(16 * (q % 8), 16)]
                pltpu.async_copy(buf[sl], acc.at[idx_v.at[kk]], ss[sl],
                                 add=True)
            for kk in range(6, 8):
                sl = kk % 2
                pltpu.make_async_copy(buf[sl], acc.at[idx_v.at[kk]],
                                      ss[sl]).wait()
            return carry

        lax.fori_loop(0, ngroups, body, 0)
        plsc.subcore_barrier()

        pltpu.sync_copy(acc.at[pl.ds(s * rows_per, rows_per)],
                        out_hbm.at[pl.ds(c * N2 + s * rows_per, rows_per)])

    return k(msg_pk, dst2)


def _tc_moments(ef_pk, W1, b1, g1, be1, E, DE, H, BR):
    EPK = E // 8
    grid = (ef_pk.shape[0] + BR - 1) // BR

    def body(ef_ref, W1_ref, b1_ref, g1_ref, be1_ref,
             m_ref, s_ref, W1s_ref, bvec_ref):
        i = pl.program_id(0)
        blk = ef_ref[...]
        rows = lax.broadcasted_iota(jnp.int32, (BR, 1), 0) + i * BR
        blk = jnp.where(rows < EPK, blk, 0.0)
        cm = jnp.zeros((DE, DE), jnp.float32)
        cs = jnp.zeros((1, DE), jnp.float32)
        for a in range(8):
            sl = blk[:, 16 * a:16 * (a + 1)]
            cm += lax.dot_general(sl, sl, (((0,), (0,)), ((), ())),
                                  preferred_element_type=jnp.float32)
            cs += jnp.sum(sl, axis=0, keepdims=True)

        @pl.when(i == 0)
        def _():
            m_ref[...] = jnp.zeros_like(m_ref)
            s_ref[...] = jnp.zeros_like(s_ref)

        m_ref[...] += cm
        s_ref[...] += cs

        @pl.when(i == grid - 1)
        def _():
            W1v = W1_ref[...]
            m = s_ref[...] / E
            C0 = m_ref[...] / E - lax.dot_general(
                m, m, (((0,), (0,)), ((), ())),
                preferred_element_type=jnp.float32)
            varh = jnp.sum(W1v * jnp.dot(C0, W1v,
                                         preferred_element_type=jnp.float32),
                           axis=0, keepdims=True)
            muh = jnp.dot(m, W1v,
                          preferred_element_type=jnp.float32) + b1_ref[...]
            scale = g1_ref[...] * lax.rsqrt(varh + EPS)
            W1s_ref[...] = W1v * scale
            bvec_ref[...] = (b1_ref[...] - muh) * scale + be1_ref[...]

    return pl.pallas_call(
        body,
        grid=(grid,),
        in_specs=[
            pl.BlockSpec((BR, 128), lambda i: (i, 0)),
            pl.BlockSpec((DE, H), lambda i: (0, 0)),
            pl.BlockSpec((1, H), lambda i: (0, 0)),
            pl.BlockSpec((1, H), lambda i: (0, 0)),
            pl.BlockSpec((1, H), lambda i: (0, 0)),
        ],
        out_specs=(
            pl.BlockSpec((DE, DE), lambda i: (0, 0)),
            pl.BlockSpec((1, DE), lambda i: (0, 0)),
            pl.BlockSpec((DE, H), lambda i: (0, 0)),
            pl.BlockSpec((1, H), lambda i: (0, 0)),
        ),
        out_shape=(
            jax.ShapeDtypeStruct((DE, DE), jnp.float32),
            jax.ShapeDtypeStruct((1, DE), jnp.float32),
            jax.ShapeDtypeStruct((DE, H), jnp.float32),
            jax.ShapeDtypeStruct((1, H), jnp.float32),
        ),
    )(ef_pk, W1, b1, g1, be1)


def _tc_fused(ef_pk, x_pk, W1s, bvec, W2, b2,
              R, S, E, Ep, DE, H, IN, OUT, BR):
    grid = Ep // 8 // BR

    def body(ef_ref, x_ref, W1s_ref, bvec_ref,
             W2_ref, b2_ref, R_ref, S_ref, out_ref):
        efp = ef_ref[...]
        xp = x_ref[...]
        ef = jnp.concatenate([efp[:, 16 * a:16 * (a + 1)] for a in range(8)],
                             axis=0)
        x = jnp.concatenate([xp[:, 16 * a:16 * (a + 1)] for a in range(8)],
                            axis=0)

        h = jnp.dot(ef, W1s_ref[...],
                    preferred_element_type=jnp.float32) + bvec_ref[...]
        h = jnp.where(h >= 0, h, 0.01 * h)
        w = jnp.dot(h, W2_ref[...],
                    preferred_element_type=jnp.float32) + b2_ref[...]
        xr = jnp.dot(x, R_ref[...], preferred_element_type=jnp.float32)
        msg = jnp.dot(w * xr, S_ref[...],
                      preferred_element_type=jnp.float32)
        out_ref[...] = jnp.concatenate(
            [msg[a * BR:(a + 1) * BR, :] for a in range(8)], axis=1)

    return pl.pallas_call(
        body,
        grid=(grid,),
        in_specs=[
            pl.BlockSpec((BR, 128), lambda i: (i, 0)),
            pl.BlockSpec((BR, 128), lambda i: (i, 0)),
            pl.BlockSpec((DE, H), lambda i: (0, 0)),
            pl.BlockSpec((1, H), lambda i: (0, 0)),
            pl.BlockSpec((H, IN * OUT), lambda i: (0, 0)),
            pl.BlockSpec((1, IN * OUT), lambda i: (0, 0)),
            pl.BlockSpec((IN, IN * OUT), lambda i: (0, 0)),
            pl.BlockSpec((IN * OUT, OUT), lambda i: (0, 0)),
        ],
        out_specs=pl.BlockSpec((BR, 128), lambda i: (i, 0)),
        out_shape=jax.ShapeDtypeStruct((Ep // 8, 128), jnp.float32),
    )(ef_pk, x_pk, W1s, bvec, W2, b2, R, S)


def _tc_final(node_feat, parts, Wroot, bias, g2, be2, N, N2, OUT):
    def body(nf_ref, p_ref, wr_ref, b_ref, g_ref, be_ref, out_ref):
        sums = p_ref[0:N, 0:OUT] + p_ref[N2:N2 + N, 0:OUT]
        cnt = p_ref[0:N, 16:17] + p_ref[N2:N2 + N, 16:17]
        aggr = sums / jnp.maximum(cnt, 1.0)
        out0 = aggr + jnp.dot(nf_ref[...], wr_ref[...],
                              preferred_element_type=jnp.float32) + b_ref[...]
        mu = jnp.mean(out0, axis=0, keepdims=True)
        var = jnp.mean((out0 - mu) ** 2, axis=0, keepdims=True)
        o = g_ref[...] * (out0 - mu) * lax.rsqrt(var + EPS) + be_ref[...]
        out_ref[...] = jnp.where(o >= 0, o, 0.01 * o)

    return pl.pallas_call(
        body,
        out_shape=jax.ShapeDtypeStruct((N, OUT), jnp.float32),
    )(node_feat, parts, Wroot, bias, g2, be2)


def kernel(node_feat, edge_feat, edge_index, batch_index,
           W1, b1, g1, be1, W2, b2, Wroot, bias, g2, be2):
    N, IN = node_feat.shape
    E, DE = edge_feat.shape
    H = W1.shape[1]
    OUT = Wroot.shape[1]

    src = edge_index[0].astype(jnp.int32)
    dst = edge_index[1].astype(jnp.int32)

    Ep = ((E + NW * CH - 1) // (NW * CH)) * (NW * CH)
    N2 = ((N + 1 + 127) // 128) * 128
    pad = jnp.arange(Ep - E, dtype=jnp.int32)
    src2 = jnp.concatenate([src, pad % N]).reshape(Ep // CH, CH)
    dst2 = jnp.concatenate(
        [dst, N + pad % (N2 - N)]).reshape(Ep // CH, CH)

    node_rep = jnp.tile(node_feat, (1, 128 // IN))
    ef_pk = jnp.concatenate(
        [edge_feat.reshape(E // 8, 8 * DE),
         jnp.zeros(((Ep - E) // 8, 8 * DE), jnp.float32)])

    R = jnp.kron(jnp.eye(IN, dtype=jnp.float32),
                 jnp.ones((1, OUT), jnp.float32))
    S = jnp.kron(jnp.ones((IN, 1), jnp.float32),
                 jnp.eye(OUT, dtype=jnp.float32))

    x_pk = _sc_gather(node_rep, src2, Ep)
    momM, moms, W1s, bvec = _tc_moments(
        ef_pk, W1, b1.reshape(1, H), g1.reshape(1, H), be1.reshape(1, H),
        E, DE, H, 512)
    msg_pk = _tc_fused(ef_pk, x_pk, W1s, bvec, W2,
                       b2.reshape(1, IN * OUT),
                       R, S, E, Ep, DE, H, IN, OUT, 512)
    parts = _sc_scatter(msg_pk, dst2, Ep, N2)
    out = _tc_final(node_feat, parts, Wroot,
                    bias.reshape(1, OUT), g2.reshape(1, OUT),
                    be2.reshape(1, OUT), N, N2, OUT)
    return out


# speedup vs baseline: 1.0537x; 1.0537x over previous
import functools

import jax
import jax.numpy as jnp
from jax import lax
from jax.experimental import pallas as pl
from jax.experimental.pallas import tpu as pltpu
from jax.experimental.pallas import tpu_sc as plsc

EPS = 1e-5
NC = 2
NS = 16
NW = NC * NS
CH = 128


def _sc_gather(node_rep, src2, Ep):
    nchunks = Ep // (NW * CH)
    ngroups = nchunks // 8
    mesh = plsc.VectorSubcoreMesh(core_axis_name="c", subcore_axis_name="s")

    @functools.partial(
        pl.kernel,
        out_type=jax.ShapeDtypeStruct((Ep // 8, 128), jnp.float32),
        mesh=mesh,
        scratch_types=[
            pltpu.VMEM((8, CH), jnp.int32),
            pltpu.VMEM((CH, 128), jnp.float32),
            pltpu.VMEM((CH, 128), jnp.float32),
            pltpu.VMEM((CH, 128), jnp.float32),
            pltpu.VMEM((CH, 128), jnp.float32),
            pltpu.VMEM((CH // 8, 128), jnp.float32),
            pltpu.VMEM((CH // 8, 128), jnp.float32),
            pltpu.SemaphoreType.DMA,
            pltpu.SemaphoreType.DMA,
            pltpu.SemaphoreType.DMA,
            pltpu.SemaphoreType.DMA,
            pltpu.SemaphoreType.DMA,
            pltpu.SemaphoreType.DMA,
        ],
    )
    def k(node_hbm, src_hbm, out_hbm, idx_v, r0, r1, r2, r3, x0, x1,
          sg0, sg1, sg2, sg3, so0, so1):
        c = lax.axis_index("c")
        s = lax.axis_index("s")
        wid = s * NC + c
        rows = (r0, r1, r2, r3)
        xb = (x0, x1)
        sg = (sg0, sg1, sg2, sg3)
        so = (so0, so1)

        def body(j, carry):
            gg = wid * ngroups + j
            pltpu.sync_copy(src_hbm.at[pl.ds(gg * 8, 8)], idx_v)
            for kk in range(3):
                pltpu.async_copy(node_hbm.at[idx_v.at[kk]],
                                 rows[kk], sg[kk])
            for kk in range(8):
                sl = kk % 4
                xs = kk % 2
                if kk < 5:
                    pltpu.async_copy(node_hbm.at[idx_v.at[kk + 3]],
                                     rows[(kk + 3) % 4], sg[(kk + 3) % 4])
                pltpu.make_async_copy(node_hbm.at[idx_v.at[kk]],
                                      rows[sl], sg[sl]).wait()
                if kk >= 2:
                    g2 = gg * 8 + kk - 2
                    pltpu.make_async_copy(
                        xb[xs],
                        out_hbm.at[pl.ds(g2 * (CH // 8), CH // 8)],
                        so[xs]).wait()
                for q in range(CH):
                    xb[xs][q // 8, pl.ds(16 * (q % 8), 16)] = \
                        rows[sl][q, pl.ds(0, 16)]
                g = gg * 8 + kk
                pltpu.async_copy(xb[xs],
                                 out_hbm.at[pl.ds(g * (CH // 8), CH // 8)],
                                 so[xs])
            for kk in range(6, 8):
                xs = kk % 2
                g = gg * 8 + kk
                pltpu.make_async_copy(
                    xb[xs], out_hbm.at[pl.ds(g * (CH // 8), CH // 8)],
                    so[xs]).wait()
            return carry

        lax.fori_loop(0, ngroups, body, 0)

    return k(node_rep, src2)


def _sc_scatter(msg_pk, dst2, Ep, N2):
    nchunks = Ep // (NW * CH)
    ngroups = nchunks // 8
    rows_per = N2 // NS
    ZR = 79
    mesh = plsc.VectorSubcoreMesh(core_axis_name="c", subcore_axis_name="s")

    @functools.partial(
        pl.kernel,
        out_type=jax.ShapeDtypeStruct((NC * N2, 128), jnp.float32),
        mesh=mesh,
        scratch_types=[
            pltpu.VMEM_SHARED((N2, 128), jnp.float32),
            pltpu.VMEM((8, CH), jnp.int32),
            pltpu.VMEM((CH // 8, 128), jnp.float32),
            pltpu.VMEM((CH // 8, 128), jnp.float32),
            pltpu.VMEM((CH, 128), jnp.float32),
            pltpu.VMEM((CH, 128), jnp.float32),
            pltpu.VMEM((ZR, 128), jnp.float32),
            pltpu.SemaphoreType.DMA,
            pltpu.SemaphoreType.DMA,
            pltpu.SemaphoreType.DMA,
            pltpu.SemaphoreType.DMA,
        ],
    )
    def k(msg_hbm, dst_hbm, out_hbm, acc, idx_v, m0, m1, b0, b1, zer,
          sm0, sm1, ss0, ss1):
        c = lax.axis_index("c")
        s = lax.axis_index("s")
        wid = s * NC + c
        mb = (m0, m1)
        buf = (b0, b1)
        sm = (sm0, sm1)
        ss = (ss0, ss1)

        zv = jnp.zeros((16,), jnp.float32)
        ov = jnp.ones((16,), jnp.float32)

        def fz(i, carry):
            for t in range(8):
                zer[i, pl.ds(16 * t, 16)] = zv
            return carry

        lax.fori_loop(0, ZR, fz, 0)

        def fb(i, carry):
            for b in buf:
                b[i, pl.ds(16, 16)] = ov
                for t in range(2, 8):
                    b[i, pl.ds(16 * t, 16)] = zv
            return carry

        lax.fori_loop(0, CH, fb, 0)

        nz = (rows_per + ZR - 1) // ZR

        def za(i, carry):
            r = jnp.minimum(i * ZR, rows_per - ZR)
            pltpu.sync_copy(zer, acc.at[pl.ds(s * rows_per + r, ZR)])
            return carry

        lax.fori_loop(0, nz, za, 0)
        plsc.subcore_barrier()

        def body(j, carry):
            gg = wid * ngroups + j
            pltpu.sync_copy(dst_hbm.at[pl.ds(gg * 8, 8)], idx_v)
            g0 = gg * 8
            pltpu.async_copy(msg_hbm.at[pl.ds(g0 * (CH // 8), CH // 8)],
                             mb[0], sm[0])
            for kk in range(8):
                sl = kk % 2
                g = gg * 8 + kk
                if kk < 7:
                    g1 = g + 1
                    pltpu.async_copy(
                        msg_hbm.at[pl.ds(g1 * (CH // 8), CH // 8)],
                        mb[(kk + 1) % 2], sm[(kk + 1) % 2])
                pltpu.make_async_copy(
                    msg_hbm.at[pl.ds(g * (CH // 8), CH // 8)],
                    mb[sl], sm[sl]).wait()
                if kk >= 2:
                    pltpu.make_async_copy(buf[sl], acc.at[idx_v.at[kk - 2]],
                                          ss[sl]).wait()
                for q in range(CH):
                    buf[sl][q, pl.ds(0, 16)] = \
                        mb[sl][q // 8, pl.ds(16 * (q % 8), 16)]
                pltpu.async_copy(buf[sl], acc.at[idx_v.at[kk]], ss[sl],
                                 add=True)
            for kk in range(6, 8):
                sl = kk % 2
                pltpu.make_async_copy(buf[sl], acc.at[idx_v.at[kk]],
                                      ss[sl]).wait()
            return carry

        lax.fori_loop(0, ngroups, body, 0)
        plsc.subcore_barrier()

        pltpu.sync_copy(acc.at[pl.ds(s * rows_per, rows_per)],
                        out_hbm.at[pl.ds(c * N2 + s * rows_per, rows_per)])

    return k(msg_pk, dst2)


def _tc_moments(ef_pk, W1, b1, g1, be1, E, DE, H, BR):
    EPK = E // 8
    grid = (ef_pk.shape[0] + BR - 1) // BR

    def body(ef_ref, W1_ref, b1_ref, g1_ref, be1_ref,
             m_ref, s_ref, W1s_ref, bvec_ref):
        i = pl.program_id(0)
        blk = ef_ref[...]
        rows = lax.broadcasted_iota(jnp.int32, (BR, 1), 0) + i * BR
        blk = jnp.where(rows < EPK, blk, 0.0)
        cm = jnp.zeros((DE, DE), jnp.float32)
        cs = jnp.zeros((1, DE), jnp.float32)
        for a in range(8):
            sl = blk[:, 16 * a:16 * (a + 1)]
            cm += lax.dot_general(sl, sl, (((0,), (0,)), ((), ())),
                                  preferred_element_type=jnp.float32)
            cs += jnp.sum(sl, axis=0, keepdims=True)

        @pl.when(i == 0)
        def _():
            m_ref[...] = jnp.zeros_like(m_ref)
            s_ref[...] = jnp.zeros_like(s_ref)

        m_ref[...] += cm
        s_ref[...] += cs

        @pl.when(i == grid - 1)
        def _():
            W1v = W1_ref[...]
            m = s_ref[...] / E
            C0 = m_ref[...] / E - lax.dot_general(
                m, m, (((0,), (0,)), ((), ())),
                preferred_element_type=jnp.float32)
            varh = jnp.sum(W1v * jnp.dot(C0, W1v,
                                         preferred_element_type=jnp.float32),
                           axis=0, keepdims=True)
            muh = jnp.dot(m, W1v,
                          preferred_element_type=jnp.float32) + b1_ref[...]
            scale = g1_ref[...] * lax.rsqrt(varh + EPS)
            W1s_ref[...] = W1v * scale
            bvec_ref[...] = (b1_ref[...] - muh) * scale + be1_ref[...]

    return pl.pallas_call(
        body,
        grid=(grid,),
        in_specs=[
            pl.BlockSpec((BR, 128), lambda i: (i, 0)),
            pl.BlockSpec((DE, H), lambda i: (0, 0)),
            pl.BlockSpec((1, H), lambda i: (0, 0)),
            pl.BlockSpec((1, H), lambda i: (0, 0)),
            pl.BlockSpec((1, H), lambda i: (0, 0)),
        ],
        out_specs=(
            pl.BlockSpec((DE, DE), lambda i: (0, 0)),
            pl.BlockSpec((1, DE), lambda i: (0, 0)),
            pl.BlockSpec((DE, H), lambda i: (0, 0)),
            pl.BlockSpec((1, H), lambda i: (0, 0)),
        ),
        out_shape=(
            jax.ShapeDtypeStruct((DE, DE), jnp.float32),
            jax.ShapeDtypeStruct((1, DE), jnp.float32),
            jax.ShapeDtypeStruct((DE, H), jnp.float32),
            jax.ShapeDtypeStruct((1, H), jnp.float32),
        ),
    )(ef_pk, W1, b1, g1, be1)


def _tc_fused(ef_pk, x_pk, W1s, bvec, W2, b2,
              R, S, E, Ep, DE, H, IN, OUT, BR):
    grid = Ep // 8 // BR

    def body(ef_ref, x_ref, W1s_ref, bvec_ref,
             W2_ref, b2_ref, R_ref, S_ref, out_ref):
        efp = ef_ref[...]
        xp = x_ref[...]
        ef = jnp.concatenate([efp[:, 16 * a:16 * (a + 1)] for a in range(8)],
                             axis=0)
        x = jnp.concatenate([xp[:, 16 * a:16 * (a + 1)] for a in range(8)],
                            axis=0)

        h = jnp.dot(ef, W1s_ref[...],
                    preferred_element_type=jnp.float32) + bvec_ref[...]
        h = jnp.where(h >= 0, h, 0.01 * h)
        w = jnp.dot(h, W2_ref[...],
                    preferred_element_type=jnp.float32) + b2_ref[...]
        xr = jnp.dot(x, R_ref[...], preferred_element_type=jnp.float32)
        msg = jnp.dot(w * xr, S_ref[...],
                      preferred_element_type=jnp.float32)
        out_ref[...] = jnp.concatenate(
            [msg[a * BR:(a + 1) * BR, :] for a in range(8)], axis=1)

    return pl.pallas_call(
        body,
        grid=(grid,),
        in_specs=[
            pl.BlockSpec((BR, 128), lambda i: (i, 0)),
            pl.BlockSpec((BR, 128), lambda i: (i, 0)),
            pl.BlockSpec((DE, H), lambda i: (0, 0)),
            pl.BlockSpec((1, H), lambda i: (0, 0)),
            pl.BlockSpec((H, IN * OUT), lambda i: (0, 0)),
            pl.BlockSpec((1, IN * OUT), lambda i: (0, 0)),
            pl.BlockSpec((IN, IN * OUT), lambda i: (0, 0)),
            pl.BlockSpec((IN * OUT, OUT), lambda i: (0, 0)),
        ],
        out_specs=pl.BlockSpec((BR, 128), lambda i: (i, 0)),
        out_shape=jax.ShapeDtypeStruct((Ep // 8, 128), jnp.float32),
    )(ef_pk, x_pk, W1s, bvec, W2, b2, R, S)


def _tc_final(node_feat, parts, Wroot, bias, g2, be2, N, N2, OUT):
    def body(nf_ref, p_ref, wr_ref, b_ref, g_ref, be_ref, out_ref):
        sums = p_ref[0:N, 0:OUT] + p_ref[N2:N2 + N, 0:OUT]
        cnt = p_ref[0:N, 16:17] + p_ref[N2:N2 + N, 16:17]
        aggr = sums / jnp.maximum(cnt, 1.0)
        out0 = aggr + jnp.dot(nf_ref[...], wr_ref[...],
                              preferred_element_type=jnp.float32) + b_ref[...]
        mu = jnp.mean(out0, axis=0, keepdims=True)
        var = jnp.mean((out0 - mu) ** 2, axis=0, keepdims=True)
        o = g_ref[...] * (out0 - mu) * lax.rsqrt(var + EPS) + be_ref[...]
        out_ref[...] = jnp.where(o >= 0, o, 0.01 * o)

    return pl.pallas_call(
        body,
        out_shape=jax.ShapeDtypeStruct((N, OUT), jnp.float32),
    )(node_feat, parts, Wroot, bias, g2, be2)


def kernel(node_feat, edge_feat, edge_index, batch_index,
           W1, b1, g1, be1, W2, b2, Wroot, bias, g2, be2):
    N, IN = node_feat.shape
    E, DE = edge_feat.shape
    H = W1.shape[1]
    OUT = Wroot.shape[1]

    src = edge_index[0].astype(jnp.int32)
    dst = edge_index[1].astype(jnp.int32)

    Ep = ((E + NW * CH - 1) // (NW * CH)) * (NW * CH)
    N2 = ((N + 1 + 127) // 128) * 128
    pad = jnp.arange(Ep - E, dtype=jnp.int32)
    src2 = jnp.concatenate([src, pad % N]).reshape(Ep // CH, CH)
    dst2 = jnp.concatenate(
        [dst, N + pad % (N2 - N)]).reshape(Ep // CH, CH)

    node_rep = jnp.tile(node_feat, (1, 128 // IN))
    ef_pk = jnp.concatenate(
        [edge_feat.reshape(E // 8, 8 * DE),
         jnp.zeros(((Ep - E) // 8, 8 * DE), jnp.float32)])

    R = jnp.kron(jnp.eye(IN, dtype=jnp.float32),
                 jnp.ones((1, OUT), jnp.float32))
    S = jnp.kron(jnp.ones((IN, 1), jnp.float32),
                 jnp.eye(OUT, dtype=jnp.float32))

    x_pk = _sc_gather(node_rep, src2, Ep)
    momM, moms, W1s, bvec = _tc_moments(
        ef_pk, W1, b1.reshape(1, H), g1.reshape(1, H), be1.reshape(1, H),
        E, DE, H, 2048)
    msg_pk = _tc_fused(ef_pk, x_pk, W1s, bvec, W2,
                       b2.reshape(1, IN * OUT),
                       R, S, E, Ep, DE, H, IN, OUT, 1024)
    parts = _sc_scatter(msg_pk, dst2, Ep, N2)
    out = _tc_final(node_feat, parts, Wroot,
                    bias.reshape(1, OUT), g2.reshape(1, OUT),
                    be2.reshape(1, OUT), N, N2, OUT)
    return out


# speedup vs baseline: 1.0808x; 1.0257x over previous
import functools

import jax
import jax.numpy as jnp
from jax import lax
from jax.experimental import pallas as pl
from jax.experimental.pallas import tpu as pltpu
from jax.experimental.pallas import tpu_sc as plsc

EPS = 1e-5
NC = 2
NS = 16
NW = NC * NS
CH = 128


def _sc_gather(node_rep, src2, Ep):
    nchunks = Ep // (NW * CH)
    ngroups = nchunks // 8
    mesh = plsc.VectorSubcoreMesh(core_axis_name="c", subcore_axis_name="s")

    @functools.partial(
        pl.kernel,
        out_type=jax.ShapeDtypeStruct((Ep // 8, 128), jnp.float32),
        mesh=mesh,
        scratch_types=[
            pltpu.VMEM((8, CH), jnp.int32),
            pltpu.VMEM((CH, 128), jnp.float32),
            pltpu.VMEM((CH, 128), jnp.float32),
            pltpu.VMEM((CH, 128), jnp.float32),
            pltpu.VMEM((CH, 128), jnp.float32),
            pltpu.VMEM((CH // 8, 128), jnp.float32),
            pltpu.VMEM((CH // 8, 128), jnp.float32),
            pltpu.SemaphoreType.DMA,
            pltpu.SemaphoreType.DMA,
            pltpu.SemaphoreType.DMA,
            pltpu.SemaphoreType.DMA,
            pltpu.SemaphoreType.DMA,
            pltpu.SemaphoreType.DMA,
        ],
    )
    def k(node_hbm, src_hbm, out_hbm, idx_v, r0, r1, r2, r3, x0, x1,
          sg0, sg1, sg2, sg3, so0, so1):
        c = lax.axis_index("c")
        s = lax.axis_index("s")
        wid = s * NC + c
        rows = (r0, r1, r2, r3)
        xb = (x0, x1)
        sg = (sg0, sg1, sg2, sg3)
        so = (so0, so1)

        def body(j, carry):
            gg = wid * ngroups + j
            pltpu.sync_copy(src_hbm.at[pl.ds(gg * 8, 8)], idx_v)
            for kk in range(3):
                pltpu.async_copy(node_hbm.at[idx_v.at[kk]],
                                 rows[kk], sg[kk])
            for kk in range(8):
                sl = kk % 4
                xs = kk % 2
                if kk < 5:
                    pltpu.async_copy(node_hbm.at[idx_v.at[kk + 3]],
                                     rows[(kk + 3) % 4], sg[(kk + 3) % 4])
                pltpu.make_async_copy(node_hbm.at[idx_v.at[kk]],
                                      rows[sl], sg[sl]).wait()
                if kk >= 2:
                    g2 = gg * 8 + kk - 2
                    pltpu.make_async_copy(
                        xb[xs],
                        out_hbm.at[pl.ds(g2 * (CH // 8), CH // 8)],
                        so[xs]).wait()
                for q in range(CH):
                    xb[xs][q // 8, pl.ds(16 * (q % 8), 16)] = \
                        rows[sl][q, pl.ds(0, 16)]
                g = gg * 8 + kk
                pltpu.async_copy(xb[xs],
                                 out_hbm.at[pl.ds(g * (CH // 8), CH // 8)],
                                 so[xs])
            for kk in range(6, 8):
                xs = kk % 2
                g = gg * 8 + kk
                pltpu.make_async_copy(
                    xb[xs], out_hbm.at[pl.ds(g * (CH // 8), CH // 8)],
                    so[xs]).wait()
            return carry

        lax.fori_loop(0, ngroups, body, 0)

    return k(node_rep, src2)


def _sc_scatter(msg_pk, dst2, nchunks, N2):
    ngroups = nchunks // 8
    rows_per = N2 // NS
    ZR = 79
    mesh = plsc.VectorSubcoreMesh(core_axis_name="c", subcore_axis_name="s")

    @functools.partial(
        pl.kernel,
        out_type=jax.ShapeDtypeStruct((NC * N2, 128), jnp.float32),
        mesh=mesh,
        scratch_types=[
            pltpu.VMEM_SHARED((N2, 128), jnp.float32),
            pltpu.VMEM((8, CH), jnp.int32),
            pltpu.VMEM((CH // 8, 128), jnp.float32),
            pltpu.VMEM((CH // 8, 128), jnp.float32),
            pltpu.VMEM((CH, 128), jnp.float32),
            pltpu.VMEM((CH, 128), jnp.float32),
            pltpu.VMEM((ZR, 128), jnp.float32),
            pltpu.SemaphoreType.DMA,
            pltpu.SemaphoreType.DMA,
            pltpu.SemaphoreType.DMA,
            pltpu.SemaphoreType.DMA,
        ],
    )
    def k(msg_hbm, dst_hbm, out_hbm, acc, idx_v, m0, m1, b0, b1, zer,
          sm0, sm1, ss0, ss1):
        c = lax.axis_index("c")
        s = lax.axis_index("s")
        wid = s * NC + c
        mb = (m0, m1)
        buf = (b0, b1)
        sm = (sm0, sm1)
        ss = (ss0, ss1)

        zv = jnp.zeros((16,), jnp.float32)
        ov = jnp.ones((16,), jnp.float32)

        def fz(i, carry):
            for t in range(8):
                zer[i, pl.ds(16 * t, 16)] = zv
            return carry

        lax.fori_loop(0, ZR, fz, 0)

        def fb(i, carry):
            for b in buf:
                b[i, pl.ds(16, 16)] = ov
                for t in range(2, 8):
                    b[i, pl.ds(16 * t, 16)] = zv
            return carry

        lax.fori_loop(0, CH, fb, 0)

        nz = (rows_per + ZR - 1) // ZR

        def za(i, carry):
            r = jnp.minimum(i * ZR, rows_per - ZR)
            pltpu.sync_copy(zer, acc.at[pl.ds(s * rows_per + r, ZR)])
            return carry

        lax.fori_loop(0, nz, za, 0)
        plsc.subcore_barrier()

        def body(j, carry):
            gg = wid * ngroups + j
            pltpu.sync_copy(dst_hbm.at[pl.ds(gg * 8, 8)], idx_v)
            g0 = gg * 8
            pltpu.async_copy(msg_hbm.at[pl.ds(g0 * (CH // 8), CH // 8)],
                             mb[0], sm[0])
            for kk in range(8):
                sl = kk % 2
                g = gg * 8 + kk
                if kk < 7:
                    g1 = g + 1
                    pltpu.async_copy(
                        msg_hbm.at[pl.ds(g1 * (CH // 8), CH // 8)],
                        mb[(kk + 1) % 2], sm[(kk + 1) % 2])
                pltpu.make_async_copy(
                    msg_hbm.at[pl.ds(g * (CH // 8), CH // 8)],
                    mb[sl], sm[sl]).wait()
                if kk >= 2:
                    pltpu.make_async_copy(buf[sl], acc.at[idx_v.at[kk - 2]],
                                          ss[sl]).wait()
                for q in range(CH):
                    buf[sl][q, pl.ds(0, 16)] = \
                        mb[sl][q // 8, pl.ds(16 * (q % 8), 16)]
                pltpu.async_copy(buf[sl], acc.at[idx_v.at[kk]], ss[sl],
                                 add=True)
            for kk in range(6, 8):
                sl = kk % 2
                pltpu.make_async_copy(buf[sl], acc.at[idx_v.at[kk]],
                                      ss[sl]).wait()
            return carry

        lax.fori_loop(0, ngroups, body, 0)
        plsc.subcore_barrier()

        pltpu.sync_copy(acc.at[pl.ds(s * rows_per, rows_per)],
                        out_hbm.at[pl.ds(c * N2 + s * rows_per, rows_per)])

    return k(msg_pk, dst2)


def _tc_moments(ef_pk, W1, b1, g1, be1, E, DE, H, BR):
    EPK = E // 8
    grid = (ef_pk.shape[0] + BR - 1) // BR

    def body(ef_ref, W1_ref, b1_ref, g1_ref, be1_ref,
             m_ref, s_ref, W1s_ref, bvec_ref):
        i = pl.program_id(0)
        blk = ef_ref[...]
        rows = lax.broadcasted_iota(jnp.int32, (BR, 1), 0) + i * BR
        blk = jnp.where(rows < EPK, blk, 0.0)
        cm = jnp.zeros((DE, DE), jnp.float32)
        cs = jnp.zeros((1, DE), jnp.float32)
        for a in range(8):
            sl = blk[:, 16 * a:16 * (a + 1)]
            cm += lax.dot_general(sl, sl, (((0,), (0,)), ((), ())),
                                  preferred_element_type=jnp.float32)
            cs += jnp.sum(sl, axis=0, keepdims=True)

        @pl.when(i == 0)
        def _():
            m_ref[...] = jnp.zeros_like(m_ref)
            s_ref[...] = jnp.zeros_like(s_ref)

        m_ref[...] += cm
        s_ref[...] += cs

        @pl.when(i == grid - 1)
        def _():
            W1v = W1_ref[...]
            m = s_ref[...] / E
            C0 = m_ref[...] / E - lax.dot_general(
                m, m, (((0,), (0,)), ((), ())),
                preferred_element_type=jnp.float32)
            varh = jnp.sum(W1v * jnp.dot(C0, W1v,
                                         preferred_element_type=jnp.float32),
                           axis=0, keepdims=True)
            muh = jnp.dot(m, W1v,
                          preferred_element_type=jnp.float32) + b1_ref[...]
            scale = g1_ref[...] * lax.rsqrt(varh + EPS)
            W1s_ref[...] = W1v * scale
            bvec_ref[...] = (b1_ref[...] - muh) * scale + be1_ref[...]

    return pl.pallas_call(
        body,
        grid=(grid,),
        in_specs=[
            pl.BlockSpec((BR, 128), lambda i: (i, 0)),
            pl.BlockSpec((DE, H), lambda i: (0, 0)),
            pl.BlockSpec((1, H), lambda i: (0, 0)),
            pl.BlockSpec((1, H), lambda i: (0, 0)),
            pl.BlockSpec((1, H), lambda i: (0, 0)),
        ],
        out_specs=(
            pl.BlockSpec((DE, DE), lambda i: (0, 0)),
            pl.BlockSpec((1, DE), lambda i: (0, 0)),
            pl.BlockSpec((DE, H), lambda i: (0, 0)),
            pl.BlockSpec((1, H), lambda i: (0, 0)),
        ),
        out_shape=(
            jax.ShapeDtypeStruct((DE, DE), jnp.float32),
            jax.ShapeDtypeStruct((1, DE), jnp.float32),
            jax.ShapeDtypeStruct((DE, H), jnp.float32),
            jax.ShapeDtypeStruct((1, H), jnp.float32),
        ),
    )(ef_pk, W1, b1, g1, be1)


def _tc_fused(ef_pk, x_pk, W1s, bvec, W2, b2,
              R, S, E, Ep, DE, H, IN, OUT, BR, off, nblk):
    grid = nblk

    def body(ef_ref, x_ref, W1s_ref, bvec_ref,
             W2_ref, b2_ref, R_ref, S_ref, out_ref):
        efp = ef_ref[...]
        xp = x_ref[...]
        ef = jnp.concatenate([efp[:, 16 * a:16 * (a + 1)] for a in range(8)],
                             axis=0)
        x = jnp.concatenate([xp[:, 16 * a:16 * (a + 1)] for a in range(8)],
                            axis=0)

        h = jnp.dot(ef, W1s_ref[...],
                    preferred_element_type=jnp.float32) + bvec_ref[...]
        h = jnp.where(h >= 0, h, 0.01 * h)
        w = jnp.dot(h, W2_ref[...],
                    preferred_element_type=jnp.float32) + b2_ref[...]
        xr = jnp.dot(x, R_ref[...], preferred_element_type=jnp.float32)
        msg = jnp.dot(w * xr, S_ref[...],
                      preferred_element_type=jnp.float32)
        out_ref[...] = jnp.concatenate(
            [msg[a * BR:(a + 1) * BR, :] for a in range(8)], axis=1)

    return pl.pallas_call(
        body,
        grid=(grid,),
        in_specs=[
            pl.BlockSpec((BR, 128), lambda i: (i + off, 0)),
            pl.BlockSpec((BR, 128), lambda i: (i + off, 0)),
            pl.BlockSpec((DE, H), lambda i: (0, 0)),
            pl.BlockSpec((1, H), lambda i: (0, 0)),
            pl.BlockSpec((H, IN * OUT), lambda i: (0, 0)),
            pl.BlockSpec((1, IN * OUT), lambda i: (0, 0)),
            pl.BlockSpec((IN, IN * OUT), lambda i: (0, 0)),
            pl.BlockSpec((IN * OUT, OUT), lambda i: (0, 0)),
        ],
        out_specs=pl.BlockSpec((BR, 128), lambda i: (i, 0)),
        out_shape=jax.ShapeDtypeStruct((nblk * BR, 128), jnp.float32),
    )(ef_pk, x_pk, W1s, bvec, W2, b2, R, S)


def _tc_final(node_feat, parts, parts2, Wroot, bias, g2, be2, N, N2, OUT):
    def body(nf_ref, p_ref, q_ref, wr_ref, b_ref, g_ref, be_ref, out_ref):
        sums = (p_ref[0:N, 0:OUT] + p_ref[N2:N2 + N, 0:OUT] +
                q_ref[0:N, 0:OUT] + q_ref[N2:N2 + N, 0:OUT])
        cnt = (p_ref[0:N, 16:17] + p_ref[N2:N2 + N, 16:17] +
               q_ref[0:N, 16:17] + q_ref[N2:N2 + N, 16:17])
        aggr = sums / jnp.maximum(cnt, 1.0)
        out0 = aggr + jnp.dot(nf_ref[...], wr_ref[...],
                              preferred_element_type=jnp.float32) + b_ref[...]
        mu = jnp.mean(out0, axis=0, keepdims=True)
        var = jnp.mean((out0 - mu) ** 2, axis=0, keepdims=True)
        o = g_ref[...] * (out0 - mu) * lax.rsqrt(var + EPS) + be_ref[...]
        out_ref[...] = jnp.where(o >= 0, o, 0.01 * o)

    return pl.pallas_call(
        body,
        out_shape=jax.ShapeDtypeStruct((N, OUT), jnp.float32),
    )(node_feat, parts, parts2, Wroot, bias, g2, be2)


def kernel(node_feat, edge_feat, edge_index, batch_index,
           W1, b1, g1, be1, W2, b2, Wroot, bias, g2, be2):
    N, IN = node_feat.shape
    E, DE = edge_feat.shape
    H = W1.shape[1]
    OUT = Wroot.shape[1]

    src = edge_index[0].astype(jnp.int32)
    dst = edge_index[1].astype(jnp.int32)

    Ep = ((E + NW * CH - 1) // (NW * CH)) * (NW * CH)
    N2 = ((N + 1 + 127) // 128) * 128
    pad = jnp.arange(Ep - E, dtype=jnp.int32)
    src2 = jnp.concatenate([src, pad % N]).reshape(Ep // CH, CH)
    dst2 = jnp.concatenate(
        [dst, N + pad % (N2 - N)]).reshape(Ep // CH, CH)

    node_rep = jnp.tile(node_feat, (1, 128 // IN))
    ef_pk = jnp.concatenate(
        [edge_feat.reshape(E // 8, 8 * DE),
         jnp.zeros(((Ep - E) // 8, 8 * DE), jnp.float32)])

    R = jnp.kron(jnp.eye(IN, dtype=jnp.float32),
                 jnp.ones((1, OUT), jnp.float32))
    S = jnp.kron(jnp.ones((IN, 1), jnp.float32),
                 jnp.eye(OUT, dtype=jnp.float32))

    x_pk = _sc_gather(node_rep, src2, Ep)
    momM, moms, W1s, bvec = _tc_moments(
        ef_pk, W1, b1.reshape(1, H), g1.reshape(1, H), be1.reshape(1, H),
        E, DE, H, 2048)
    BR = 1024
    nblk = Ep // 8 // BR
    nblk1 = (nblk * 3) // 5
    ch1 = nblk1 * BR * 8 // CH
    nc1 = ch1 // NW
    nc2 = (Ep // CH - ch1) // NW
    msg1 = _tc_fused(ef_pk, x_pk, W1s, bvec, W2, b2.reshape(1, IN * OUT),
                     R, S, E, Ep, DE, H, IN, OUT, BR, 0, nblk1)
    parts1 = _sc_scatter(msg1, dst2[:ch1], nc1, N2)
    msg2 = _tc_fused(ef_pk, x_pk, W1s, bvec, W2, b2.reshape(1, IN * OUT),
                     R, S, E, Ep, DE, H, IN, OUT, BR, nblk1, nblk - nblk1)
    parts2 = _sc_scatter(msg2, dst2[ch1:], nc2, N2)
    out = _tc_final(node_feat, parts1, parts2, Wroot,
                    bias.reshape(1, OUT), g2.reshape(1, OUT),
                    be2.reshape(1, OUT), N, N2, OUT)
    return out
